# baseline (speedup 1.0000x reference)
import functools

import jax
import jax.numpy as jnp
from jax import lax
from jax.experimental import pallas as pl
from jax.experimental.pallas import tpu as pltpu
from jax.experimental.pallas import tpu_sc as plsc

N = 10000
H = 128
E = 320000
NLAYER = 3
NGRAPH = 128

NC = 2
NS = 16
NW = NC * NS
EPW = E // NW
CH = 72
NCH = 140
IB = 20
NIB = NCH // IB
NBUF = 4
EPWP = NCH * CH
AGG_ROWS = EPWP

BM = 1000
NBLK = N // BM


def _sc_aggregate_body(h_hbm, src_hbm, dst_hbm, out_hbm,
                       rows_v, src_v, dst_v, agg_sh,
                       sg0, sg1, sg2, sg3, ss0, ss1, ss2, ss3):
    semg = [sg0, sg1, sg2, sg3]
    sems = [ss0, ss1, ss2, ss3]
    c = lax.axis_index("c")
    s = lax.axis_index("s")
    wid = s * NC + c

    zeros16 = jnp.zeros((16,), jnp.float32)

    def _zero_row(i, carry):
        for k in range(H // 16):
            rows_v[0, i, pl.ds(k * 16, 16)] = zeros16
        return carry

    lax.fori_loop(0, CH, _zero_row, 0)
    base = s * (AGG_ROWS // NS)

    @pl.when(s < NS - 1)
    def _():
        def _zero_chunk(q, carry):
            pltpu.sync_copy(rows_v.at[0], agg_sh.at[pl.ds(base + q * CH, CH)])
            return carry

        lax.fori_loop(0, 9, _zero_chunk, 0)

    @pl.when(s == NS - 1)
    def _():
        def _zero_chunk(q, carry):
            pltpu.sync_copy(rows_v.at[0], agg_sh.at[pl.ds(base + q * CH, CH)])
            return carry

        lax.fori_loop(0, 8, _zero_chunk, 0)
        pltpu.sync_copy(rows_v.at[0, pl.ds(0, 54)],
                        agg_sh.at[pl.ds(base + 8 * CH, 54)])

    plsc.subcore_barrier()

    pltpu.sync_copy(src_hbm.at[wid, 0], src_v.at[0])
    pltpu.sync_copy(dst_hbm.at[wid, 0], dst_v.at[0])
    for b0 in range(NBUF - 1):
        pltpu.async_copy(h_hbm.at[src_v.at[0, b0]], rows_v.at[b0], semg[b0])

    def _block(m, carry):
        sl = lax.rem(m, 2)

        @pl.when(m < NIB - 1)
        def _():
            pltpu.sync_copy(src_hbm.at[wid, m + 1], src_v.at[1 - sl])
            pltpu.sync_copy(dst_hbm.at[wid, m + 1], dst_v.at[1 - sl])

        for b in range(IB):
            buf = b % NBUF
            bufp = (b + NBUF - 1) % NBUF
            bn = b + NBUF - 1
            pltpu.make_async_copy(h_hbm.at[src_v.at[sl, b]], rows_v.at[buf],
                                  semg[buf]).wait()
            pltpu.async_copy(rows_v.at[buf], agg_sh.at[dst_v.at[sl, b]],
                             sems[buf], add=True)

            def _drain_and_prefetch(slp, bp, sln, bnn):
                pltpu.make_async_copy(
                    rows_v.at[bufp], agg_sh.at[dst_v.at[slp, bp]],
                    sems[bufp]).wait()
                pltpu.async_copy(h_hbm.at[src_v.at[sln, bnn]],
                                 rows_v.at[bufp], semg[bufp])

            if b == 0:
                @pl.when(m > 0)
                def _():
                    _drain_and_prefetch(1 - sl, IB - 1, sl, bn)

                @pl.when(m == 0)
                def _():
                    pltpu.async_copy(h_hbm.at[src_v.at[sl, bn]],
                                     rows_v.at[bufp], semg[bufp])
            elif bn < IB:
                _drain_and_prefetch(sl, b - 1, sl, bn)
            else:
                @pl.when(m < NIB - 1)
                def _():
                    _drain_and_prefetch(sl, b - 1, 1 - sl, bn - IB)

                @pl.when(m == NIB - 1)
                def _():
                    pltpu.make_async_copy(
                        rows_v.at[bufp], agg_sh.at[dst_v.at[sl, b - 1]],
                        sems[bufp]).wait()

        return carry

    lax.fori_loop(0, NIB, _block, 0)
    pltpu.make_async_copy(rows_v.at[(IB - 1) % NBUF],
                          agg_sh.at[dst_v.at[(NIB - 1) % 2, IB - 1]],
                          sems[(IB - 1) % NBUF]).wait()
    plsc.subcore_barrier()

    @pl.when(s < NS - 1)
    def _():
        pltpu.sync_copy(agg_sh.at[pl.ds(s * 624, 624)],
                        out_hbm.at[c, pl.ds(s * 624, 624)])

    @pl.when(s == NS - 1)
    def _():
        pltpu.sync_copy(agg_sh.at[pl.ds((NS - 1) * 624, 640)],
                        out_hbm.at[c, pl.ds((NS - 1) * 624, 640)])


@functools.lru_cache(maxsize=1)
def _get_sc_aggregate():
    return pl.kernel(
        _sc_aggregate_body,
        mesh=plsc.VectorSubcoreMesh(core_axis_name="c", subcore_axis_name="s"),
        out_type=jax.ShapeDtypeStruct((NC, N, H), jnp.float32),
        scratch_types=[
            pltpu.VMEM((NBUF, CH, H), jnp.float32),
            pltpu.VMEM((2, IB, CH), jnp.int32),
            pltpu.VMEM((2, IB, CH), jnp.int32),
            pltpu.VMEM_SHARED((AGG_ROWS, H), jnp.float32),
        ] + [pltpu.SemaphoreType.DMA] * (2 * NBUF),
    )


def _mlp_body(h_ref, a0_ref, a1_ref, w1_ref, b1_ref, w2_ref, b2_ref, o_ref):
    z = h_ref[...] + a0_ref[...] + a1_ref[...]
    t = jnp.maximum(
        jnp.dot(z, w1_ref[...], preferred_element_type=jnp.float32)
        + b1_ref[...], 0.0)
    o_ref[...] = jnp.maximum(
        jnp.dot(t, w2_ref[...], preferred_element_type=jnp.float32)
        + b2_ref[...], 0.0)


def _mlp_call(h, a0, a1, w1, b1, w2, b2):
    blk = pl.BlockSpec((BM, H), lambda i: (i, 0))
    wblk = pl.BlockSpec((H, H), lambda i: (0, 0))
    bblk = pl.BlockSpec((1, H), lambda i: (0, 0))
    return pl.pallas_call(
        _mlp_body,
        grid=(NBLK,),
        in_specs=[blk, blk, blk, wblk, bblk, wblk, bblk],
        out_specs=blk,
        out_shape=jax.ShapeDtypeStruct((N, H), jnp.float32),
    )(h, a0, a1, w1, b1, w2, b2)


def _mlp_pool_body(h_ref, a0_ref, a1_ref, w1_ref, b1_ref, w2_ref, b2_ref,
                   bt_ref, o_ref):
    i = pl.program_id(0)
    z = h_ref[...] + a0_ref[...] + a1_ref[...]
    t = jnp.maximum(
        jnp.dot(z, w1_ref[...], preferred_element_type=jnp.float32)
        + b1_ref[...], 0.0)
    h3 = jnp.maximum(
        jnp.dot(t, w2_ref[...], preferred_element_type=jnp.float32)
        + b2_ref[...], 0.0)
    b = bt_ref[0, 0, :]
    oh = (b[:, None] == lax.broadcasted_iota(jnp.int32, (BM, NGRAPH), 1)
          ).astype(jnp.float32)
    pooled = lax.dot_general(oh, h3, (((0,), (0,)), ((), ())),
                             preferred_element_type=jnp.float32)

    @pl.when(i == 0)
    def _():
        o_ref[...] = pooled

    @pl.when(i > 0)
    def _():
        o_ref[...] += pooled

    @pl.when(i == pl.num_programs(0) - 1)
    def _():
        p = o_ref[...]
        m = jnp.max(p, axis=1, keepdims=True)
        lse = jnp.log(jnp.sum(jnp.exp(p - m), axis=1, keepdims=True))
        o_ref[...] = p - m - lse


def _mlp_pool_call(h, a0, a1, w1, b1, w2, b2, batch3):
    blk = pl.BlockSpec((BM, H), lambda i: (i, 0))
    wblk = pl.BlockSpec((H, H), lambda i: (0, 0))
    bblk = pl.BlockSpec((1, H), lambda i: (0, 0))
    btblk = pl.BlockSpec((1, 1, BM), lambda i: (i, 0, 0))
    oblk = pl.BlockSpec((NGRAPH, NGRAPH), lambda i: (0, 0))
    return pl.pallas_call(
        _mlp_pool_body,
        grid=(NBLK,),
        in_specs=[blk, blk, blk, wblk, bblk, wblk, bblk, btblk],
        out_specs=oblk,
        out_shape=jax.ShapeDtypeStruct((NGRAPH, NGRAPH), jnp.float32),
    )(h, a0, a1, w1, b1, w2, b2, batch3)


def kernel(x, edge_index, batch, W1, b1, W2, b2):
    src = edge_index[0]
    dst = edge_index[1]
    pad = EPWP - EPW
    src_p = jnp.concatenate(
        [src.reshape(NW, EPW), jnp.zeros((NW, pad), jnp.int32)], axis=1
    ).reshape(NW, NIB, IB, CH)
    dst_p = jnp.concatenate(
        [dst.reshape(NW, EPW), jnp.full((NW, pad), N, jnp.int32)], axis=1
    ).reshape(NW, NIB, IB, CH)
    batch3 = batch.reshape(NBLK, 1, BM)

    h = x
    for l in range(NLAYER):
        agg = _get_sc_aggregate()(h, src_p, dst_p)
        w1 = W1[l]
        b1l = b1[l].reshape(1, H)
        w2 = W2[l]
        b2l = b2[l].reshape(1, H)
        if l < NLAYER - 1:
            h = _mlp_call(h, agg[0], agg[1], w1, b1l, w2, b2l)
        else:
            out = _mlp_pool_call(h, agg[0], agg[1], w1, b1l, w2, b2l, batch3)
    return out


# speedup vs baseline: 1.0251x; 1.0251x over previous
import functools

import jax
import jax.numpy as jnp
from jax import lax
from jax.experimental import pallas as pl
from jax.experimental.pallas import tpu as pltpu
from jax.experimental.pallas import tpu_sc as plsc

N = 10000
H = 128
E = 320000
NLAYER = 3
NGRAPH = 128

NC = 2
NS = 16
NW = NC * NS
EPW = E // NW
CH = 96
NCH = 105
IB = 21
NIB = NCH // IB
NBUF = 3
EPWP = NCH * CH
AGG_ROWS = EPWP

BM = 1000
NBLK = N // BM


def _sc_aggregate_body(h_hbm, src_hbm, dst_hbm, out_hbm,
                       rows_v, src_v, dst_v, agg_sh,
                       sg0, sg1, sg2, ss0, ss1, ss2):
    semg = [sg0, sg1, sg2]
    sems = [ss0, ss1, ss2]
    c = lax.axis_index("c")
    s = lax.axis_index("s")
    wid = s * NC + c

    pltpu.sync_copy(src_hbm.at[wid, 0], src_v.at[0])
    pltpu.sync_copy(dst_hbm.at[wid, 0], dst_v.at[0])
    for b0 in range(NBUF - 1):
        pltpu.async_copy(h_hbm.at[src_v.at[0, b0]], rows_v.at[b0], semg[b0])

    zeros16 = jnp.zeros((16,), jnp.float32)
    zb = NBUF - 1

    def _zero_row(i, carry):
        for k in range(H // 16):
            rows_v[zb, i, pl.ds(k * 16, 16)] = zeros16
        return carry

    lax.fori_loop(0, CH, _zero_row, 0)
    base = s * (AGG_ROWS // NS)

    @pl.when(s < NS - 1)
    def _():
        def _zero_chunk(q, carry):
            pltpu.sync_copy(rows_v.at[zb], agg_sh.at[pl.ds(base + q * CH, CH)])
            return carry

        lax.fori_loop(0, 7, _zero_chunk, 0)

    @pl.when(s == NS - 1)
    def _():
        def _zero_chunk(q, carry):
            pltpu.sync_copy(rows_v.at[zb], agg_sh.at[pl.ds(base + q * CH, CH)])
            return carry

        lax.fori_loop(0, 6, _zero_chunk, 0)
        pltpu.sync_copy(rows_v.at[zb, pl.ds(0, 54)],
                        agg_sh.at[pl.ds(base + 6 * CH, 54)])

    plsc.subcore_barrier()


    def _block(m, carry):
        sl = lax.rem(m, 2)

        @pl.when(m < NIB - 1)
        def _():
            pltpu.sync_copy(src_hbm.at[wid, m + 1], src_v.at[1 - sl])
            pltpu.sync_copy(dst_hbm.at[wid, m + 1], dst_v.at[1 - sl])

        for b in range(IB):
            buf = b % NBUF
            bufp = (b + NBUF - 1) % NBUF
            bn = b + NBUF - 1
            pltpu.make_async_copy(h_hbm.at[src_v.at[sl, b]], rows_v.at[buf],
                                  semg[buf]).wait()
            pltpu.async_copy(rows_v.at[buf], agg_sh.at[dst_v.at[sl, b]],
                             sems[buf], add=True)

            def _drain_and_prefetch(slp, bp, sln, bnn):
                pltpu.make_async_copy(
                    rows_v.at[bufp], agg_sh.at[dst_v.at[slp, bp]],
                    sems[bufp]).wait()
                pltpu.async_copy(h_hbm.at[src_v.at[sln, bnn]],
                                 rows_v.at[bufp], semg[bufp])

            if b == 0:
                @pl.when(m > 0)
                def _():
                    _drain_and_prefetch(1 - sl, IB - 1, sl, bn)

                @pl.when(m == 0)
                def _():
                    pltpu.async_copy(h_hbm.at[src_v.at[sl, bn]],
                                     rows_v.at[bufp], semg[bufp])
            elif bn < IB:
                _drain_and_prefetch(sl, b - 1, sl, bn)
            else:
                @pl.when(m < NIB - 1)
                def _():
                    _drain_and_prefetch(sl, b - 1, 1 - sl, bn - IB)

                @pl.when(m == NIB - 1)
                def _():
                    pltpu.make_async_copy(
                        rows_v.at[bufp], agg_sh.at[dst_v.at[sl, b - 1]],
                        sems[bufp]).wait()

        return carry

    lax.fori_loop(0, NIB, _block, 0)
    pltpu.make_async_copy(rows_v.at[(IB - 1) % NBUF],
                          agg_sh.at[dst_v.at[(NIB - 1) % 2, IB - 1]],
                          sems[(IB - 1) % NBUF]).wait()
    plsc.subcore_barrier()

    @pl.when(s < NS - 1)
    def _():
        pltpu.sync_copy(agg_sh.at[pl.ds(s * 624, 624)],
                        out_hbm.at[c, pl.ds(s * 624, 624)])

    @pl.when(s == NS - 1)
    def _():
        pltpu.sync_copy(agg_sh.at[pl.ds((NS - 1) * 624, 640)],
                        out_hbm.at[c, pl.ds((NS - 1) * 624, 640)])


@functools.lru_cache(maxsize=1)
def _get_sc_aggregate():
    return pl.kernel(
        _sc_aggregate_body,
        mesh=plsc.VectorSubcoreMesh(core_axis_name="c", subcore_axis_name="s"),
        out_type=jax.ShapeDtypeStruct((NC, N, H), jnp.float32),
        scratch_types=[
            pltpu.VMEM((NBUF, CH, H), jnp.float32),
            pltpu.VMEM((2, IB, CH), jnp.int32),
            pltpu.VMEM((2, IB, CH), jnp.int32),
            pltpu.VMEM_SHARED((AGG_ROWS, H), jnp.float32),
        ] + [pltpu.SemaphoreType.DMA] * (2 * NBUF),
    )


def _mlp_body(h_ref, a0_ref, a1_ref, w1_ref, b1_ref, w2_ref, b2_ref, o_ref):
    z = h_ref[...] + a0_ref[...] + a1_ref[...]
    t = jnp.maximum(
        jnp.dot(z, w1_ref[...], preferred_element_type=jnp.float32)
        + b1_ref[...], 0.0)
    o_ref[...] = jnp.maximum(
        jnp.dot(t, w2_ref[...], preferred_element_type=jnp.float32)
        + b2_ref[...], 0.0)


def _mlp_call(h, a0, a1, w1, b1, w2, b2):
    blk = pl.BlockSpec((BM, H), lambda i: (i, 0))
    wblk = pl.BlockSpec((H, H), lambda i: (0, 0))
    bblk = pl.BlockSpec((1, H), lambda i: (0, 0))
    return pl.pallas_call(
        _mlp_body,
        grid=(NBLK,),
        in_specs=[blk, blk, blk, wblk, bblk, wblk, bblk],
        out_specs=blk,
        out_shape=jax.ShapeDtypeStruct((N, H), jnp.float32),
    )(h, a0, a1, w1, b1, w2, b2)


def _mlp_pool_body(h_ref, a0_ref, a1_ref, w1_ref, b1_ref, w2_ref, b2_ref,
                   bt_ref, o_ref):
    i = pl.program_id(0)
    z = h_ref[...] + a0_ref[...] + a1_ref[...]
    t = jnp.maximum(
        jnp.dot(z, w1_ref[...], preferred_element_type=jnp.float32)
        + b1_ref[...], 0.0)
    h3 = jnp.maximum(
        jnp.dot(t, w2_ref[...], preferred_element_type=jnp.float32)
        + b2_ref[...], 0.0)
    b = bt_ref[0, 0, :]
    oh = (b[:, None] == lax.broadcasted_iota(jnp.int32, (BM, NGRAPH), 1)
          ).astype(jnp.float32)
    pooled = lax.dot_general(oh, h3, (((0,), (0,)), ((), ())),
                             preferred_element_type=jnp.float32)

    @pl.when(i == 0)
    def _():
        o_ref[...] = pooled

    @pl.when(i > 0)
    def _():
        o_ref[...] += pooled

    @pl.when(i == pl.num_programs(0) - 1)
    def _():
        p = o_ref[...]
        m = jnp.max(p, axis=1, keepdims=True)
        lse = jnp.log(jnp.sum(jnp.exp(p - m), axis=1, keepdims=True))
        o_ref[...] = p - m - lse


def _mlp_pool_call(h, a0, a1, w1, b1, w2, b2, batch3):
    blk = pl.BlockSpec((BM, H), lambda i: (i, 0))
    wblk = pl.BlockSpec((H, H), lambda i: (0, 0))
    bblk = pl.BlockSpec((1, H), lambda i: (0, 0))
    btblk = pl.BlockSpec((1, 1, BM), lambda i: (i, 0, 0))
    oblk = pl.BlockSpec((NGRAPH, NGRAPH), lambda i: (0, 0))
    return pl.pallas_call(
        _mlp_pool_body,
        grid=(NBLK,),
        in_specs=[blk, blk, blk, wblk, bblk, wblk, bblk, btblk],
        out_specs=oblk,
        out_shape=jax.ShapeDtypeStruct((NGRAPH, NGRAPH), jnp.float32),
    )(h, a0, a1, w1, b1, w2, b2, batch3)


def kernel(x, edge_index, batch, W1, b1, W2, b2):
    src = edge_index[0]
    dst = edge_index[1]
    pad = EPWP - EPW
    src_p = jnp.concatenate(
        [src.reshape(NW, EPW), jnp.zeros((NW, pad), jnp.int32)], axis=1
    ).reshape(NW, NIB, IB, CH)
    dst_p = jnp.concatenate(
        [dst.reshape(NW, EPW), jnp.full((NW, pad), N, jnp.int32)], axis=1
    ).reshape(NW, NIB, IB, CH)
    batch3 = batch.reshape(NBLK, 1, BM)

    h = x
    for l in range(NLAYER):
        agg = _get_sc_aggregate()(h, src_p, dst_p)
        w1 = W1[l]
        b1l = b1[l].reshape(1, H)
        w2 = W2[l]
        b2l = b2[l].reshape(1, H)
        if l < NLAYER - 1:
            h = _mlp_call(h, agg[0], agg[1], w1, b1l, w2, b2l)
        else:
            out = _mlp_pool_call(h, agg[0], agg[1], w1, b1l, w2, b2l, batch3)
    return out


# speedup vs baseline: 1.0413x; 1.0158x over previous
import functools

import jax
import jax.numpy as jnp
from jax import lax
from jax.experimental import pallas as pl
from jax.experimental.pallas import tpu as pltpu
from jax.experimental.pallas import tpu_sc as plsc

N = 10000
H = 128
E = 320000
NLAYER = 3
NGRAPH = 128

NC = 2
NS = 16
NW = NC * NS
EPW = E // NW
CH = 96
NCH = 105
IB = 21
NIB = NCH // IB
NBUF = 3
EPWP = NCH * CH
AGG_ROWS = EPWP

BM = 2000
NBLK = N // BM


def _sc_aggregate_body(h_hbm, src_hbm, dst_hbm, out_hbm,
                       rows_v, src_v, dst_v, agg_sh,
                       sg0, sg1, sg2, ss0, ss1, ss2):
    semg = [sg0, sg1, sg2]
    sems = [ss0, ss1, ss2]
    c = lax.axis_index("c")
    s = lax.axis_index("s")
    wid = s * NC + c

    pltpu.sync_copy(src_hbm.at[wid, 0], src_v.at[0])
    pltpu.sync_copy(dst_hbm.at[wid, 0], dst_v.at[0])
    for b0 in range(NBUF - 1):
        pltpu.async_copy(h_hbm.at[src_v.at[0, b0]], rows_v.at[b0], semg[b0])

    zeros16 = jnp.zeros((16,), jnp.float32)
    zb = NBUF - 1

    def _zero_row(i, carry):
        for k in range(H // 16):
            rows_v[zb, i, pl.ds(k * 16, 16)] = zeros16
        return carry

    lax.fori_loop(0, CH, _zero_row, 0)
    base = s * (AGG_ROWS // NS)

    @pl.when(s < NS - 1)
    def _():
        def _zero_chunk(q, carry):
            pltpu.sync_copy(rows_v.at[zb], agg_sh.at[pl.ds(base + q * CH, CH)])
            return carry

        lax.fori_loop(0, 7, _zero_chunk, 0)

    @pl.when(s == NS - 1)
    def _():
        def _zero_chunk(q, carry):
            pltpu.sync_copy(rows_v.at[zb], agg_sh.at[pl.ds(base + q * CH, CH)])
            return carry

        lax.fori_loop(0, 6, _zero_chunk, 0)
        pltpu.sync_copy(rows_v.at[zb, pl.ds(0, 54)],
                        agg_sh.at[pl.ds(base + 6 * CH, 54)])

    plsc.subcore_barrier()


    def _block(m, carry):
        sl = lax.rem(m, 2)

        @pl.when(m < NIB - 1)
        def _():
            pltpu.sync_copy(src_hbm.at[wid, m + 1], src_v.at[1 - sl])
            pltpu.sync_copy(dst_hbm.at[wid, m + 1], dst_v.at[1 - sl])

        for b in range(IB):
            buf = b % NBUF
            bufp = (b + NBUF - 1) % NBUF
            bn = b + NBUF - 1
            pltpu.make_async_copy(h_hbm.at[src_v.at[sl, b]], rows_v.at[buf],
                                  semg[buf]).wait()
            pltpu.async_copy(rows_v.at[buf], agg_sh.at[dst_v.at[sl, b]],
                             sems[buf], add=True)

            def _drain_and_prefetch(slp, bp, sln, bnn):
                pltpu.make_async_copy(
                    rows_v.at[bufp], agg_sh.at[dst_v.at[slp, bp]],
                    sems[bufp]).wait()
                pltpu.async_copy(h_hbm.at[src_v.at[sln, bnn]],
                                 rows_v.at[bufp], semg[bufp])

            if b == 0:
                @pl.when(m > 0)
                def _():
                    _drain_and_prefetch(1 - sl, IB - 1, sl, bn)

                @pl.when(m == 0)
                def _():
                    pltpu.async_copy(h_hbm.at[src_v.at[sl, bn]],
                                     rows_v.at[bufp], semg[bufp])
            elif bn < IB:
                _drain_and_prefetch(sl, b - 1, sl, bn)
            else:
                @pl.when(m < NIB - 1)
                def _():
                    _drain_and_prefetch(sl, b - 1, 1 - sl, bn - IB)

                @pl.when(m == NIB - 1)
                def _():
                    pltpu.make_async_copy(
                        rows_v.at[bufp], agg_sh.at[dst_v.at[sl, b - 1]],
                        sems[bufp]).wait()

        return carry

    lax.fori_loop(0, NIB, _block, 0)
    pltpu.make_async_copy(rows_v.at[(IB - 1) % NBUF],
                          agg_sh.at[dst_v.at[(NIB - 1) % 2, IB - 1]],
                          sems[(IB - 1) % NBUF]).wait()
    plsc.subcore_barrier()

    @pl.when(s < NS - 1)
    def _():
        pltpu.sync_copy(agg_sh.at[pl.ds(s * 624, 624)],
                        out_hbm.at[c, pl.ds(s * 624, 624)])

    @pl.when(s == NS - 1)
    def _():
        pltpu.sync_copy(agg_sh.at[pl.ds((NS - 1) * 624, 640)],
                        out_hbm.at[c, pl.ds((NS - 1) * 624, 640)])


@functools.lru_cache(maxsize=1)
def _get_sc_aggregate():
    return pl.kernel(
        _sc_aggregate_body,
        mesh=plsc.VectorSubcoreMesh(core_axis_name="c", subcore_axis_name="s"),
        out_type=jax.ShapeDtypeStruct((NC, N, H), jnp.float32),
        scratch_types=[
            pltpu.VMEM((NBUF, CH, H), jnp.float32),
            pltpu.VMEM((2, IB, CH), jnp.int32),
            pltpu.VMEM((2, IB, CH), jnp.int32),
            pltpu.VMEM_SHARED((AGG_ROWS, H), jnp.float32),
        ] + [pltpu.SemaphoreType.DMA] * (2 * NBUF),
    )


def _mlp_body(h_ref, a0_ref, a1_ref, w1_ref, b1_ref, w2_ref, b2_ref, o_ref):
    z = h_ref[...] + a0_ref[...] + a1_ref[...]
    t = jnp.maximum(
        jnp.dot(z, w1_ref[...], preferred_element_type=jnp.float32)
        + b1_ref[...], 0.0)
    o_ref[...] = jnp.maximum(
        jnp.dot(t, w2_ref[...], preferred_element_type=jnp.float32)
        + b2_ref[...], 0.0)


def _mlp_call(h, a0, a1, w1, b1, w2, b2):
    blk = pl.BlockSpec((BM, H), lambda i: (i, 0))
    wblk = pl.BlockSpec((H, H), lambda i: (0, 0))
    bblk = pl.BlockSpec((1, H), lambda i: (0, 0))
    return pl.pallas_call(
        _mlp_body,
        grid=(NBLK,),
        in_specs=[blk, blk, blk, wblk, bblk, wblk, bblk],
        out_specs=blk,
        out_shape=jax.ShapeDtypeStruct((N, H), jnp.float32),
    )(h, a0, a1, w1, b1, w2, b2)


def _mlp_pool_body(h_ref, a0_ref, a1_ref, w1_ref, b1_ref, w2_ref, b2_ref,
                   bt_ref, o_ref):
    i = pl.program_id(0)
    z = h_ref[...] + a0_ref[...] + a1_ref[...]
    t = jnp.maximum(
        jnp.dot(z, w1_ref[...], preferred_element_type=jnp.float32)
        + b1_ref[...], 0.0)
    h3 = jnp.maximum(
        jnp.dot(t, w2_ref[...], preferred_element_type=jnp.float32)
        + b2_ref[...], 0.0)
    b = bt_ref[0, 0, :]
    oh = (b[:, None] == lax.broadcasted_iota(jnp.int32, (BM, NGRAPH), 1)
          ).astype(jnp.float32)
    pooled = lax.dot_general(oh, h3, (((0,), (0,)), ((), ())),
                             preferred_element_type=jnp.float32)

    @pl.when(i == 0)
    def _():
        o_ref[...] = pooled

    @pl.when(i > 0)
    def _():
        o_ref[...] += pooled

    @pl.when(i == pl.num_programs(0) - 1)
    def _():
        p = o_ref[...]
        m = jnp.max(p, axis=1, keepdims=True)
        lse = jnp.log(jnp.sum(jnp.exp(p - m), axis=1, keepdims=True))
        o_ref[...] = p - m - lse


def _mlp_pool_call(h, a0, a1, w1, b1, w2, b2, batch3):
    blk = pl.BlockSpec((BM, H), lambda i: (i, 0))
    wblk = pl.BlockSpec((H, H), lambda i: (0, 0))
    bblk = pl.BlockSpec((1, H), lambda i: (0, 0))
    btblk = pl.BlockSpec((1, 1, BM), lambda i: (i, 0, 0))
    oblk = pl.BlockSpec((NGRAPH, NGRAPH), lambda i: (0, 0))
    return pl.pallas_call(
        _mlp_pool_body,
        grid=(NBLK,),
        in_specs=[blk, blk, blk, wblk, bblk, wblk, bblk, btblk],
        out_specs=oblk,
        out_shape=jax.ShapeDtypeStruct((NGRAPH, NGRAPH), jnp.float32),
    )(h, a0, a1, w1, b1, w2, b2, batch3)


def kernel(x, edge_index, batch, W1, b1, W2, b2):
    src = edge_index[0]
    dst = edge_index[1]
    pad = EPWP - EPW
    src_p = jnp.concatenate(
        [src.reshape(NW, EPW), jnp.zeros((NW, pad), jnp.int32)], axis=1
    ).reshape(NW, NIB, IB, CH)
    dst_p = jnp.concatenate(
        [dst.reshape(NW, EPW), jnp.full((NW, pad), N, jnp.int32)], axis=1
    ).reshape(NW, NIB, IB, CH)
    batch3 = batch.reshape(NBLK, 1, BM)

    h = x
    for l in range(NLAYER):
        agg = _get_sc_aggregate()(h, src_p, dst_p)
        w1 = W1[l]
        b1l = b1[l].reshape(1, H)
        w2 = W2[l]
        b2l = b2[l].reshape(1, H)
        if l < NLAYER - 1:
            h = _mlp_call(h, agg[0], agg[1], w1, b1l, w2, b2l)
        else:
            out = _mlp_pool_call(h, agg[0], agg[1], w1, b1l, w2, b2l, batch3)
    return out


# speedup vs baseline: 1.0452x; 1.0037x over previous
import functools

import jax
import jax.numpy as jnp
from jax import lax
from jax.experimental import pallas as pl
from jax.experimental.pallas import tpu as pltpu
from jax.experimental.pallas import tpu_sc as plsc

N = 10000
H = 128
E = 320000
NLAYER = 3
NGRAPH = 128

NC = 2
NS = 16
NW = NC * NS
EPW = E // NW
CH = 96
NCH = 105
IB = 21
NIB = NCH // IB
NBUF = 3
EPWP = NCH * CH
AGG_ROWS = EPWP

BM = 5000
NBLK = N // BM


def _sc_aggregate_body(h_hbm, src_hbm, dst_hbm, out_hbm,
                       rows_v, src_v, dst_v, agg_sh,
                       sg0, sg1, sg2, ss0, ss1, ss2):
    semg = [sg0, sg1, sg2]
    sems = [ss0, ss1, ss2]
    c = lax.axis_index("c")
    s = lax.axis_index("s")
    wid = s * NC + c

    pltpu.sync_copy(src_hbm.at[wid, 0], src_v.at[0])
    pltpu.sync_copy(dst_hbm.at[wid, 0], dst_v.at[0])
    for b0 in range(NBUF - 1):
        pltpu.async_copy(h_hbm.at[src_v.at[0, b0]], rows_v.at[b0], semg[b0])

    zeros16 = jnp.zeros((16,), jnp.float32)
    zb = NBUF - 1

    def _zero_row(i, carry):
        for k in range(H // 16):
            rows_v[zb, i, pl.ds(k * 16, 16)] = zeros16
        return carry

    lax.fori_loop(0, CH, _zero_row, 0)
    base = s * (AGG_ROWS // NS)

    @pl.when(s < NS - 1)
    def _():
        def _zero_chunk(q, carry):
            pltpu.sync_copy(rows_v.at[zb], agg_sh.at[pl.ds(base + q * CH, CH)])
            return carry

        lax.fori_loop(0, 7, _zero_chunk, 0)

    @pl.when(s == NS - 1)
    def _():
        def _zero_chunk(q, carry):
            pltpu.sync_copy(rows_v.at[zb], agg_sh.at[pl.ds(base + q * CH, CH)])
            return carry

        lax.fori_loop(0, 6, _zero_chunk, 0)
        pltpu.sync_copy(rows_v.at[zb, pl.ds(0, 54)],
                        agg_sh.at[pl.ds(base + 6 * CH, 54)])

    plsc.subcore_barrier()


    def _block(m, carry):
        sl = lax.rem(m, 2)

        @pl.when(m < NIB - 1)
        def _():
            pltpu.sync_copy(src_hbm.at[wid, m + 1], src_v.at[1 - sl])
            pltpu.sync_copy(dst_hbm.at[wid, m + 1], dst_v.at[1 - sl])

        for b in range(IB):
            buf = b % NBUF
            bufp = (b + NBUF - 1) % NBUF
            bn = b + NBUF - 1
            pltpu.make_async_copy(h_hbm.at[src_v.at[sl, b]], rows_v.at[buf],
                                  semg[buf]).wait()
            pltpu.async_copy(rows_v.at[buf], agg_sh.at[dst_v.at[sl, b]],
                             sems[buf], add=True)

            def _drain_and_prefetch(slp, bp, sln, bnn):
                pltpu.make_async_copy(
                    rows_v.at[bufp], agg_sh.at[dst_v.at[slp, bp]],
                    sems[bufp]).wait()
                pltpu.async_copy(h_hbm.at[src_v.at[sln, bnn]],
                                 rows_v.at[bufp], semg[bufp])

            if b == 0:
                @pl.when(m > 0)
                def _():
                    _drain_and_prefetch(1 - sl, IB - 1, sl, bn)

                @pl.when(m == 0)
                def _():
                    pltpu.async_copy(h_hbm.at[src_v.at[sl, bn]],
                                     rows_v.at[bufp], semg[bufp])
            elif bn < IB:
                _drain_and_prefetch(sl, b - 1, sl, bn)
            else:
                @pl.when(m < NIB - 1)
                def _():
                    _drain_and_prefetch(sl, b - 1, 1 - sl, bn - IB)

                @pl.when(m == NIB - 1)
                def _():
                    pltpu.make_async_copy(
                        rows_v.at[bufp], agg_sh.at[dst_v.at[sl, b - 1]],
                        sems[bufp]).wait()

        return carry

    lax.fori_loop(0, NIB, _block, 0)
    pltpu.make_async_copy(rows_v.at[(IB - 1) % NBUF],
                          agg_sh.at[dst_v.at[(NIB - 1) % 2, IB - 1]],
                          sems[(IB - 1) % NBUF]).wait()
    plsc.subcore_barrier()

    @pl.when(s < NS - 1)
    def _():
        pltpu.sync_copy(agg_sh.at[pl.ds(s * 624, 624)],
                        out_hbm.at[c, pl.ds(s * 624, 624)])

    @pl.when(s == NS - 1)
    def _():
        pltpu.sync_copy(agg_sh.at[pl.ds((NS - 1) * 624, 640)],
                        out_hbm.at[c, pl.ds((NS - 1) * 624, 640)])


@functools.lru_cache(maxsize=1)
def _get_sc_aggregate():
    return pl.kernel(
        _sc_aggregate_body,
        mesh=plsc.VectorSubcoreMesh(core_axis_name="c", subcore_axis_name="s"),
        out_type=jax.ShapeDtypeStruct((NC, N, H), jnp.float32),
        scratch_types=[
            pltpu.VMEM((NBUF, CH, H), jnp.float32),
            pltpu.VMEM((2, IB, CH), jnp.int32),
            pltpu.VMEM((2, IB, CH), jnp.int32),
            pltpu.VMEM_SHARED((AGG_ROWS, H), jnp.float32),
        ] + [pltpu.SemaphoreType.DMA] * (2 * NBUF),
    )


def _mlp_body(h_ref, a0_ref, a1_ref, w1_ref, b1_ref, w2_ref, b2_ref, o_ref):
    z = h_ref[...] + a0_ref[...] + a1_ref[...]
    t = jnp.maximum(
        jnp.dot(z, w1_ref[...], preferred_element_type=jnp.float32)
        + b1_ref[...], 0.0)
    o_ref[...] = jnp.maximum(
        jnp.dot(t, w2_ref[...], preferred_element_type=jnp.float32)
        + b2_ref[...], 0.0)


def _mlp_call(h, a0, a1, w1, b1, w2, b2):
    blk = pl.BlockSpec((BM, H), lambda i: (i, 0))
    wblk = pl.BlockSpec((H, H), lambda i: (0, 0))
    bblk = pl.BlockSpec((1, H), lambda i: (0, 0))
    return pl.pallas_call(
        _mlp_body,
        grid=(NBLK,),
        in_specs=[blk, blk, blk, wblk, bblk, wblk, bblk],
        out_specs=blk,
        out_shape=jax.ShapeDtypeStruct((N, H), jnp.float32),
    )(h, a0, a1, w1, b1, w2, b2)


def _mlp_pool_body(h_ref, a0_ref, a1_ref, w1_ref, b1_ref, w2_ref, b2_ref,
                   bt_ref, o_ref):
    i = pl.program_id(0)
    z = h_ref[...] + a0_ref[...] + a1_ref[...]
    t = jnp.maximum(
        jnp.dot(z, w1_ref[...], preferred_element_type=jnp.float32)
        + b1_ref[...], 0.0)
    h3 = jnp.maximum(
        jnp.dot(t, w2_ref[...], preferred_element_type=jnp.float32)
        + b2_ref[...], 0.0)
    b = bt_ref[0, 0, :]
    oh = (b[:, None] == lax.broadcasted_iota(jnp.int32, (BM, NGRAPH), 1)
          ).astype(jnp.float32)
    pooled = lax.dot_general(oh, h3, (((0,), (0,)), ((), ())),
                             preferred_element_type=jnp.float32)

    @pl.when(i == 0)
    def _():
        o_ref[...] = pooled

    @pl.when(i > 0)
    def _():
        o_ref[...] += pooled

    @pl.when(i == pl.num_programs(0) - 1)
    def _():
        p = o_ref[...]
        m = jnp.max(p, axis=1, keepdims=True)
        lse = jnp.log(jnp.sum(jnp.exp(p - m), axis=1, keepdims=True))
        o_ref[...] = p - m - lse


def _mlp_pool_call(h, a0, a1, w1, b1, w2, b2, batch3):
    blk = pl.BlockSpec((BM, H), lambda i: (i, 0))
    wblk = pl.BlockSpec((H, H), lambda i: (0, 0))
    bblk = pl.BlockSpec((1, H), lambda i: (0, 0))
    btblk = pl.BlockSpec((1, 1, BM), lambda i: (i, 0, 0))
    oblk = pl.BlockSpec((NGRAPH, NGRAPH), lambda i: (0, 0))
    return pl.pallas_call(
        _mlp_pool_body,
        grid=(NBLK,),
        in_specs=[blk, blk, blk, wblk, bblk, wblk, bblk, btblk],
        out_specs=oblk,
        out_shape=jax.ShapeDtypeStruct((NGRAPH, NGRAPH), jnp.float32),
    )(h, a0, a1, w1, b1, w2, b2, batch3)


def kernel(x, edge_index, batch, W1, b1, W2, b2):
    src = edge_index[0]
    dst = edge_index[1]
    pad = EPWP - EPW
    src_p = jnp.concatenate(
        [src.reshape(NW, EPW), jnp.zeros((NW, pad), jnp.int32)], axis=1
    ).reshape(NW, NIB, IB, CH)
    dst_p = jnp.concatenate(
        [dst.reshape(NW, EPW), jnp.full((NW, pad), N, jnp.int32)], axis=1
    ).reshape(NW, NIB, IB, CH)
    batch3 = batch.reshape(NBLK, 1, BM)

    h = x
    for l in range(NLAYER):
        agg = _get_sc_aggregate()(h, src_p, dst_p)
        w1 = W1[l]
        b1l = b1[l].reshape(1, H)
        w2 = W2[l]
        b2l = b2[l].reshape(1, H)
        if l < NLAYER - 1:
            h = _mlp_call(h, agg[0], agg[1], w1, b1l, w2, b2l)
        else:
            out = _mlp_pool_call(h, agg[0], agg[1], w1, b1l, w2, b2l, batch3)
    return out
